# Initial kernel scaffold; baseline (speedup 1.0000x reference)
import jax
import jax.numpy as jnp
from jax.experimental import pallas as pl


def kernel(feat, edge_index, weight, bias):
    raise NotImplementedError("write your pallas kernel here")


# speedup vs baseline: 6.5550x; 6.5550x over previous
import functools

import jax
import jax.numpy as jnp
from jax import lax
from jax.experimental import pallas as pl
from jax.experimental.pallas import tpu as pltpu
from jax.experimental.pallas import tpu_sc as plsc

NC = 2
NS = 16
CH = 80


def _mesh():
    return plsc.VectorSubcoreMesh(
        core_axis_name="c", subcore_axis_name="s", num_cores=NC, num_subcores=NS
    )


def _degrees_call(ei_deg, zeros_row, ones_row, np_pad, n_chunks):
    rows_per_tile = np_pad // NS

    def body(ei_hbm, z_hbm, ones_hbm, degs_hbm, acc, idxb, ones_v, stage):
        c = lax.axis_index("c")
        s = lax.axis_index("s")
        off = pl.multiple_of(s * rows_per_tile, 8)
        pltpu.sync_copy(z_hbm, stage)
        pltpu.sync_copy(stage, acc.at[pl.ds(off, rows_per_tile)])
        pltpu.sync_copy(ei_hbm.at[c, s], idxb)
        pltpu.sync_copy(ones_hbm, ones_v)
        plsc.subcore_barrier()

        def step(i, carry):
            pltpu.sync_copy(ones_v, acc.at[idxb.at[i]], add=True)
            return carry

        lax.fori_loop(0, n_chunks, step, 0)
        plsc.subcore_barrier()
        out_off = pl.multiple_of(c * np_pad + s * rows_per_tile, 8)
        pltpu.sync_copy(acc.at[pl.ds(off, rows_per_tile)], stage)
        pltpu.sync_copy(stage, degs_hbm.at[pl.ds(out_off, rows_per_tile)])

    call = pl.kernel(
        body,
        out_type=jax.ShapeDtypeStruct((2 * np_pad,), jnp.float32),
        mesh=_mesh(),
        compiler_params=pltpu.CompilerParams(use_tc_tiling_on_sc=False),
        scratch_types=[
            pltpu.VMEM_SHARED((np_pad,), jnp.float32),
            pltpu.VMEM((n_chunks, CH), jnp.int32),
            pltpu.VMEM((CH,), jnp.float32),
            pltpu.VMEM((np_pad // NS,), jnp.float32),
        ],
    )
    return call(ei_deg, zeros_row, ones_row)


def _aggregate_call(feat_src, ei_agg, zeros_blk, np_pad, n_chunks):
    d = feat_src.shape[1]
    rows_per_tile = np_pad // NS

    n_sub = rows_per_tile // CH

    def body(feat_hbm, ei_hbm, z_hbm, parts_hbm, acc, sidx, didx, rows, sem):
        c = lax.axis_index("c")
        s = lax.axis_index("s")
        off = pl.multiple_of(s * rows_per_tile, 8)
        pltpu.sync_copy(z_hbm, rows)
        for k in range(n_sub):
            pltpu.sync_copy(rows, acc.at[pl.ds(off + k * CH, CH)])
        pltpu.sync_copy(ei_hbm.at[0, c, s], sidx)
        pltpu.sync_copy(ei_hbm.at[1, c, s], didx)
        plsc.subcore_barrier()

        def step(i, carry):
            pltpu.async_copy(feat_hbm.at[sidx.at[i]], rows, sem).wait()
            pltpu.sync_copy(rows, acc.at[didx.at[i]], add=True)
            return carry

        lax.fori_loop(0, n_chunks, step, 0)
        plsc.subcore_barrier()
        for k in range(n_sub):
            pltpu.sync_copy(acc.at[pl.ds(off + k * CH, CH)], rows)
            pltpu.sync_copy(rows, parts_hbm.at[c, pl.ds(off + k * CH, CH)])

    call = pl.kernel(
        body,
        out_type=jax.ShapeDtypeStruct((2, np_pad, d), jnp.float32),
        mesh=_mesh(),
        compiler_params=pltpu.CompilerParams(use_tc_tiling_on_sc=False),
        scratch_types=[
            pltpu.VMEM_SHARED((np_pad, d), jnp.float32),
            pltpu.VMEM((n_chunks, CH), jnp.int32),
            pltpu.VMEM((n_chunks, CH), jnp.int32),
            pltpu.VMEM((CH, d), jnp.float32),
            pltpu.SemaphoreType.DMA,
        ],
    )
    return call(feat_src, ei_agg, zeros_blk)


def kernel(feat, edge_index, weight, bias):
    n, d = feat.shape
    e = edge_index.shape[1]
    np_pad = -(-n // (NS * 64)) * (NS * 64)
    ept_deg = e // NS
    ept_agg = e // (NC * NS)
    nck_deg = ept_deg // CH
    nck_agg = ept_agg // CH

    ei_deg = edge_index.reshape(2, NS, nck_deg, CH)
    ei_agg = edge_index.reshape(2, NC, NS, nck_agg, CH)
    zeros_row = jnp.zeros((np_pad // NS,), jnp.float32)
    ones_row = jnp.ones((CH,), jnp.float32)
    zeros_blk = jnp.zeros((CH, d), jnp.float32)

    degs = _degrees_call(ei_deg, zeros_row, ones_row, np_pad, nck_deg)
    degs = degs.reshape(2, np_pad)

    def scale_body(feat_ref, degs_ref, out_ref):
        od = degs_ref[0, :n]
        out_ref[...] = feat_ref[...] * lax.rsqrt(jnp.maximum(od, 1.0))[:, None]

    feat_src = pl.pallas_call(
        scale_body,
        out_shape=jax.ShapeDtypeStruct((n, d), jnp.float32),
    )(feat, degs)

    parts = _aggregate_call(feat_src, ei_agg, zeros_blk, np_pad, nck_agg)

    def fin_body(parts_ref, degs_ref, w_ref, b_ref, out_ref):
        m = parts_ref[0, :n, :] + parts_ref[1, :n, :]
        r = jnp.dot(m, w_ref[...], preferred_element_type=jnp.float32)
        si = lax.rsqrt(jnp.maximum(degs_ref[1, :n], 1.0))
        out_ref[...] = r * si[:, None] + b_ref[...]

    return pl.pallas_call(
        fin_body,
        out_shape=jax.ShapeDtypeStruct((n, d), jnp.float32),
    )(parts, degs, weight, bias.reshape(1, d))
